# baseline (speedup 1.0000x reference)
import functools

import jax
import jax.numpy as jnp
from jax import lax
from jax.experimental import pallas as pl
from jax.experimental.pallas import tpu as pltpu
from jax.experimental.pallas import tpu_sc as plsc

EMBED = 64
LANES = 16
NC = 2
NS = 16
NW = NC * NS

STEP = 256
POS_REP = 64


def _build(ntok):
    rows_per_w = ntok // NW
    nsteps = rows_per_w // STEP
    mesh = plsc.VectorSubcoreMesh(core_axis_name="c", subcore_axis_name="s")

    @functools.partial(
        pl.kernel,
        out_type=jax.ShapeDtypeStruct((ntok, EMBED), jnp.float32),
        mesh=mesh,
        compiler_params=pltpu.CompilerParams(use_tc_tiling_on_sc=False),
        scratch_types=[
            pltpu.VMEM((rows_per_w,), jnp.int32),
            pltpu.VMEM((rows_per_w,), jnp.int32),
            pltpu.VMEM((STEP, EMBED), jnp.float32),
            pltpu.VMEM((STEP, EMBED), jnp.float32),
            pltpu.VMEM((STEP, EMBED), jnp.float32),
            pltpu.VMEM((STEP, EMBED), jnp.float32),
            pltpu.SemaphoreType.DMA,
            pltpu.SemaphoreType.DMA,
            pltpu.SemaphoreType.DMA,
            pltpu.SemaphoreType.DMA,
            pltpu.SemaphoreType.DMA,
            pltpu.SemaphoreType.DMA,
        ],
    )
    def emb_kernel(tok_hbm, pidx_hbm, table_hbm, pos_hbm, out_hbm,
                   tok_v, pidx_v, rows0, rows1, prows0, prows1,
                   sgt0, sgt1, sgp0, sgp1, so0, so1):
        wid = lax.axis_index("s") * NC + lax.axis_index("c")
        w_base = wid * rows_per_w
        rows = (rows0, rows1)
        prows = (prows0, prows1)
        sgt = (sgt0, sgt1)
        sgp = (sgp0, sgp1)
        so = (so0, so1)

        pltpu.async_copy(tok_hbm.at[pl.ds(w_base, rows_per_w)], tok_v, sgt0)
        pltpu.async_copy(pidx_hbm.at[pl.ds(w_base, rows_per_w)], pidx_v, sgp0)
        pltpu.make_async_copy(tok_hbm.at[pl.ds(w_base, rows_per_w)], tok_v,
                              sgt0).wait()
        pltpu.make_async_copy(pidx_hbm.at[pl.ds(w_base, rows_per_w)], pidx_v,
                              sgp0).wait()

        def issue_gathers(st, b):
            off = pl.multiple_of(st * STEP, 8)
            pltpu.async_copy(table_hbm.at[tok_v.at[pl.ds(off, STEP)]],
                             rows[b], sgt[b])
            pltpu.async_copy(pos_hbm.at[pidx_v.at[pl.ds(off, STEP)]],
                             prows[b], sgp[b])

        def wait_gathers(b):
            pltpu.make_async_copy(table_hbm.at[tok_v.at[pl.ds(0, STEP)]],
                                  rows[b], sgt[b]).wait()
            pltpu.make_async_copy(pos_hbm.at[pidx_v.at[pl.ds(0, STEP)]],
                                  prows[b], sgp[b]).wait()

        def wait_out(b):
            pltpu.make_async_copy(rows[b], out_hbm.at[pl.ds(0, STEP)],
                                  so[b]).wait()

        issue_gathers(0, 0)

        def pair_body(j, carry):
            for b in range(2):
                st = 2 * j + b
                nb = 1 - b

                @pl.when(st + 1 < nsteps)
                def _issue_next():
                    @pl.when(st >= 1)
                    def _drain_out():
                        wait_out(nb)
                    issue_gathers(st + 1, nb)

                wait_gathers(b)

                @plsc.parallel_loop(0, STEP, unroll=8)
                def _row_body(r):
                    for k in range(EMBED // LANES):
                        sl = pl.ds(k * LANES, LANES)
                        rows[b][r, sl] = rows[b][r, sl] + prows[b][r, sl]

                base = pl.multiple_of(w_base + st * STEP, 8)
                pltpu.async_copy(rows[b], out_hbm.at[pl.ds(base, STEP)], so[b])
            return carry

        lax.fori_loop(0, nsteps // 2, pair_body, 0)
        wait_out(0)
        wait_out(1)

    return emb_kernel


def kernel(x, padding_mask, table, pos_enc):
    b, s = x.shape
    ntok = b * s
    tok = x.reshape(ntok).astype(jnp.int32)
    s_ids = jnp.arange(s, dtype=jnp.int32)[None, :]
    prows = pos_enc.shape[0] + 1
    pidx = jnp.where(padding_mask, jnp.int32(prows - 1), s_ids).reshape(ntok)
    rep = (jnp.arange(ntok, dtype=jnp.int32) % POS_REP) * prows
    pidx = pidx + rep
    pos_ext = jnp.concatenate(
        [pos_enc.astype(jnp.float32),
         jnp.zeros((1, pos_enc.shape[1]), jnp.float32)], axis=0)
    pos_rep = jnp.tile(pos_ext, (POS_REP, 1))
    out = _build(ntok)(tok, pidx, table, pos_rep)
    return out.reshape(b, s, EMBED)
